# baseline (speedup 1.0000x reference)
import jax
import jax.numpy as jnp
from jax.experimental import pallas as pl
from jax.experimental.pallas import tpu as pltpu

_E_LIST = [(0, 1), (1, 3), (3, 1), (1, 4), (4, 1), (1, 0),
           (0, 2), (2, 5), (5, 2), (2, 6), (6, 2), (2, 0)]
_ROOT = 0
_T_RECUR = 2
_N_NODES = 7
_H = 256
_C = 128
_B = 4096
_N_STREAMS = 2
_LOAD_ORDER = [0, 1, 3, 4, 2, 5, 6]
_OV_SLOTS = 4


def _mmt(a, w):
    return jax.lax.dot_general(
        a, w, (((1,), (1,)), ((), ())), preferred_element_type=jnp.float32)


def _walk_kernel(x_hbm, wm_ref, wu_ref, wc_ref, bm_ref, bu_ref, bc_ref,
                 out_hbm, xv, ov, load_sem, store_sem):
    for i in _LOAD_ORDER:
        pltpu.make_async_copy(x_hbm.at[i], xv.at[i], load_sem.at[i]).start()

    wm = wm_ref[...]
    wu = wu_ref[...]
    wc = wc_ref[...]
    bu = bu_ref[...]
    bmu = bm_ref[...] + bu
    bc = bc_ref[...]

    part = _B // _N_STREAMS
    loaded = set()
    pending = {}

    def need(*nodes):
        for n in nodes:
            if n not in loaded:
                pltpu.make_async_copy(
                    x_hbm.at[n], xv.at[n], load_sem.at[n]).wait()
                loaded.add(n)

    def emit(e, states):
        slot = e % _OV_SLOTS
        if slot in pending:
            pltpu.make_async_copy(
                ov.at[slot], out_hbm.at[pending[slot]],
                store_sem.at[slot]).wait()
        for s, st in enumerate(states):
            ov[slot, s * part:(s + 1) * part] = _mmt(st, wc) + bc
        pltpu.make_async_copy(
            ov.at[slot], out_hbm.at[e], store_sem.at[slot]).start()
        pending[slot] = e

    need(_ROOT)
    h = [None] * _N_NODES
    h[_ROOT] = [xv[_ROOT, s * part:(s + 1) * part] for s in range(_N_STREAMS)]
    emit(0, h[_ROOT])
    for e, (u, v) in enumerate(_E_LIST):
        need(u, v)
        for n in (u, v):
            if h[n] is None:
                h[n] = [xv[n, s * part:(s + 1) * part]
                        for s in range(_N_STREAMS)]
        hus = [jnp.tanh(_mmt(h[u][s], wu) + _mmt(h[v][s], wm) + bmu)
               for s in range(_N_STREAMS)]
        for _ in range(_T_RECUR):
            hus = [jnp.tanh(_mmt(hu, wu) + bu) for hu in hus]
        h[u] = hus
        emit(e + 1, hus)

    for slot, e in pending.items():
        pltpu.make_async_copy(
            ov.at[slot], out_hbm.at[e], store_sem.at[slot]).wait()


def kernel(x, W_msg, b_msg, W_upd, b_upd, W_cls, b_cls):
    n_out = 1 + len(_E_LIST)
    return pl.pallas_call(
        _walk_kernel,
        grid=(),
        in_specs=[
            pl.BlockSpec(memory_space=pl.ANY),
            pl.BlockSpec(memory_space=pltpu.MemorySpace.VMEM),
            pl.BlockSpec(memory_space=pltpu.MemorySpace.VMEM),
            pl.BlockSpec(memory_space=pltpu.MemorySpace.VMEM),
            pl.BlockSpec(memory_space=pltpu.MemorySpace.VMEM),
            pl.BlockSpec(memory_space=pltpu.MemorySpace.VMEM),
            pl.BlockSpec(memory_space=pltpu.MemorySpace.VMEM),
        ],
        out_specs=pl.BlockSpec(memory_space=pl.ANY),
        out_shape=jax.ShapeDtypeStruct((n_out, _B, _C), jnp.float32),
        scratch_shapes=[
            pltpu.MemorySpace.VMEM((_N_NODES, _B, _H), jnp.float32),
            pltpu.MemorySpace.VMEM((_OV_SLOTS, _B, _C), jnp.float32),
            pltpu.SemaphoreType.DMA((_N_NODES,)),
            pltpu.SemaphoreType.DMA((_OV_SLOTS,)),
        ],
        compiler_params=pltpu.CompilerParams(
            vmem_limit_bytes=120 * 1024 * 1024),
    )(x.transpose(1, 0, 2), W_msg, W_upd, W_cls,
      b_msg.reshape(1, _H), b_upd.reshape(1, _H), b_cls.reshape(1, _C))


# speedup vs baseline: 1.0809x; 1.0809x over previous
import jax
import jax.numpy as jnp
from jax.experimental import pallas as pl
from jax.experimental.pallas import tpu as pltpu

_E_LIST = [(0, 1), (1, 3), (3, 1), (1, 4), (4, 1), (1, 0),
           (0, 2), (2, 5), (5, 2), (2, 6), (6, 2), (2, 0)]
_ROOT = 0
_T_RECUR = 2
_N_NODES = 7
_H = 256
_C = 128
_TB = 1024
_N_STREAMS = 2


def _mmt(a, w):
    return jax.lax.dot_general(
        a, w, (((1,), (1,)), ((), ())), preferred_element_type=jnp.float32)


def _walk_kernel(x_ref, wm_ref, wu_ref, wc_ref, bm_ref, bu_ref, bc_ref,
                 out_ref):
    wm = wm_ref[...]
    wu = wu_ref[...]
    wc = wc_ref[...]
    bu = bu_ref[...]
    bmu = bm_ref[...] + bu
    bc = bc_ref[...]

    part = _TB // _N_STREAMS
    streams = [
        [x_ref[i, s * part:(s + 1) * part] for i in range(_N_NODES)]
        for s in range(_N_STREAMS)
    ]
    for s, h in enumerate(streams):
        out_ref[0, s * part:(s + 1) * part] = _mmt(h[_ROOT], wc) + bc
    for e, (u, v) in enumerate(_E_LIST):
        hus = [jnp.tanh(_mmt(h[u], wu) + _mmt(h[v], wm) + bmu)
               for h in streams]
        for _ in range(_T_RECUR):
            hus = [jnp.tanh(_mmt(hu, wu) + bu) for hu in hus]
        for s, h in enumerate(streams):
            h[u] = hus[s]
            out_ref[e + 1, s * part:(s + 1) * part] = _mmt(hus[s], wc) + bc


def kernel(x, W_msg, b_msg, W_upd, b_upd, W_cls, b_cls):
    B = x.shape[0]
    n_out = 1 + len(_E_LIST)
    grid = (B // _TB,)
    return pl.pallas_call(
        _walk_kernel,
        grid=grid,
        in_specs=[
            pl.BlockSpec((_N_NODES, _TB, _H), lambda i: (0, i, 0)),
            pl.BlockSpec((_H, _H), lambda i: (0, 0)),
            pl.BlockSpec((_H, _H), lambda i: (0, 0)),
            pl.BlockSpec((_C, _H), lambda i: (0, 0)),
            pl.BlockSpec((1, _H), lambda i: (0, 0)),
            pl.BlockSpec((1, _H), lambda i: (0, 0)),
            pl.BlockSpec((1, _C), lambda i: (0, 0)),
        ],
        out_specs=pl.BlockSpec((n_out, _TB, _C), lambda i: (0, i, 0)),
        out_shape=jax.ShapeDtypeStruct((n_out, B, _C), jnp.float32),
        compiler_params=pltpu.CompilerParams(
            vmem_limit_bytes=100 * 1024 * 1024),
    )(x.transpose(1, 0, 2), W_msg, W_upd, W_cls,
      b_msg.reshape(1, _H), b_upd.reshape(1, _H), b_cls.reshape(1, _C))
